# baseline (speedup 1.0000x reference)
import jax
import jax.numpy as jnp
from jax.experimental import pallas as pl
from jax.experimental.pallas import tpu as pltpu

N_NODES = 10000
N_EDGES = 8192
FEATS = 32
ALPHA = 0.5
BETA = 0.5

NC = 2
BN1 = 200
NI1 = N_NODES // (NC * BN1)
BN2 = 1000
NI2 = N_NODES // (NC * BN2)

F8 = jnp.float8_e4m3fn
YS = 4096.0
DS = 0.0625

_NT = (((1,), (0,)), ((), ()))
_PARAMS = pltpu.CompilerParams(dimension_semantics=("parallel", "arbitrary"))


def _crow(acc):
    return jax.lax.rsqrt(acc[FEATS:FEATS + 1, :] * (1.0 / DS)
                         * acc[FEATS + 1:FEATS + 2, :])


def _stats_sweep(x_ref, inc_ref, acc_ref, dv_ref, incq_ref):
    i = pl.program_id(1)
    inc = inc_ref[...]
    incq_ref[...] = inc.astype(F8)
    dvb = jnp.sum(inc, axis=1, keepdims=True)
    dv_ref[...] = dvb

    @pl.when(i == 0)
    def _init():
        acc_ref[...] = jnp.zeros_like(acc_ref)

    lhs = jnp.concatenate(
        [x_ref[...].T.astype(F8),
         (dvb.T * DS).astype(F8),
         jnp.ones((1, BN1), F8)], axis=0)
    acc_ref[0] += jax.lax.dot_general(
        lhs, incq_ref[...], _NT, preferred_element_type=jnp.float32)


def _layer1_sweep(incq_ref, x_ref, dv_ref, acc_ref, w1_ref,
                  m2t_ref, y_ref):
    i = pl.program_id(1)

    @pl.when(i == 0)
    def _start():
        acc = acc_ref[0] + acc_ref[1]
        y_ref[...] = (acc[0:FEATS, :] * (_crow(acc) * YS)).T.astype(F8)
        m2t_ref[...] = jnp.zeros_like(m2t_ref)

    x0p = jax.lax.dot_general(
        incq_ref[...], y_ref[...], _NT, preferred_element_type=jnp.float32)
    xcombt = (((1.0 - ALPHA) / YS) * x0p.T * jax.lax.rsqrt(dv_ref[...].T)
              + ALPHA * x_ref[...].T)
    xlt = (1.0 - BETA) * xcombt + BETA * jax.lax.dot_general(
        w1_ref[...], xcombt, _NT, preferred_element_type=jnp.float32)
    m2t_ref[0] += jax.lax.dot_general(
        xlt.astype(F8), incq_ref[...], _NT,
        preferred_element_type=jnp.float32)


def _layer2_sweep(incq_ref, x_ref, dv_ref, acc_ref, m2t_ref, w2_ref, wo_ref,
                  b_ref, out_ref, y_ref):
    i = pl.program_id(1)

    @pl.when(i == 0)
    def _start():
        acc = acc_ref[0] + acc_ref[1]
        m2t = m2t_ref[0] + m2t_ref[1]
        y_ref[...] = (m2t * (_crow(acc) * YS)).T.astype(F8)

    x0p = jax.lax.dot_general(
        incq_ref[...], y_ref[...], _NT, preferred_element_type=jnp.float32)
    xcombt = (((1.0 - ALPHA) / YS) * x0p.T * jax.lax.rsqrt(dv_ref[...].T)
              + ALPHA * x_ref[...].T)
    xlt = (1.0 - BETA) * xcombt + BETA * jax.lax.dot_general(
        w2_ref[...], xcombt, _NT, preferred_element_type=jnp.float32)
    outt = jax.lax.dot_general(
        wo_ref[...], xlt, _NT, preferred_element_type=jnp.float32)
    out_ref[...] = outt.T + b_ref[...]


def kernel(x_0, incidence_1, W_layers, W_out, b_out):
    n, e, f = N_NODES, N_EDGES, FEATS
    full = lambda shape: pl.BlockSpec(shape, lambda *_: (0,) * len(shape))

    acc2, dv, inc_q = pl.pallas_call(
        _stats_sweep,
        grid=(NC, NI1),
        in_specs=[pl.BlockSpec((BN1, f), lambda c, i: (c * NI1 + i, 0)),
                  pl.BlockSpec((BN1, e), lambda c, i: (c * NI1 + i, 0))],
        out_specs=[pl.BlockSpec((1, f + 2, e), lambda c, i: (c, 0, 0)),
                   pl.BlockSpec((BN1, 1), lambda c, i: (c * NI1 + i, 0)),
                   pl.BlockSpec((BN1, e), lambda c, i: (c * NI1 + i, 0))],
        out_shape=[
            jax.ShapeDtypeStruct((NC, f + 2, e), jnp.float32),
            jax.ShapeDtypeStruct((n, 1), jnp.float32),
            jax.ShapeDtypeStruct((n, e), F8),
        ],
        compiler_params=_PARAMS,
    )(x_0, incidence_1)

    m2t2 = pl.pallas_call(
        _layer1_sweep,
        grid=(NC, NI2),
        in_specs=[pl.BlockSpec((BN2, e), lambda c, i: (c * NI2 + i, 0)),
                  pl.BlockSpec((BN2, f), lambda c, i: (c * NI2 + i, 0)),
                  pl.BlockSpec((BN2, 1), lambda c, i: (c * NI2 + i, 0)),
                  full((NC, f + 2, e)), full((f, f))],
        out_specs=pl.BlockSpec((1, f, e), lambda c, i: (c, 0, 0)),
        out_shape=jax.ShapeDtypeStruct((NC, f, e), jnp.float32),
        scratch_shapes=[pltpu.VMEM((e, f), F8)],
        compiler_params=_PARAMS,
    )(inc_q, x_0, dv, acc2, W_layers[0])

    out = pl.pallas_call(
        _layer2_sweep,
        grid=(NC, NI2),
        in_specs=[pl.BlockSpec((BN2, e), lambda c, i: (c * NI2 + i, 0)),
                  pl.BlockSpec((BN2, f), lambda c, i: (c * NI2 + i, 0)),
                  pl.BlockSpec((BN2, 1), lambda c, i: (c * NI2 + i, 0)),
                  full((NC, f + 2, e)), full((NC, f, e)), full((f, f)),
                  full((f, f)), full((1, f))],
        out_specs=pl.BlockSpec((BN2, f), lambda c, i: (c * NI2 + i, 0)),
        out_shape=jax.ShapeDtypeStruct((n, f), jnp.float32),
        scratch_shapes=[pltpu.VMEM((e, f), F8)],
        compiler_params=_PARAMS,
    )(inc_q, x_0, dv, acc2, m2t2, W_layers[1], W_out, b_out[None, :])

    return out


# speedup vs baseline: 1.0707x; 1.0707x over previous
import jax
import jax.numpy as jnp
from jax.experimental import pallas as pl
from jax.experimental.pallas import tpu as pltpu

N_NODES = 10000
N_EDGES = 8192
FEATS = 32
ALPHA = 0.5
BETA = 0.5

BN1 = 400
BN2 = 2000

F8 = jnp.float8_e4m3fn
YS = 4096.0
DS = 0.0625

_NT = (((1,), (0,)), ((), ()))


def _stats_sweep(x_ref, inc_ref, acc_ref, dv_ref, incq_ref):
    i = pl.program_id(0)
    inc = inc_ref[...]
    incq = inc.astype(F8)
    incq_ref[...] = incq
    dvb = jnp.sum(inc, axis=1, keepdims=True)
    dv_ref[...] = dvb

    @pl.when(i == 0)
    def _init():
        acc_ref[...] = jnp.zeros_like(acc_ref)

    lhs = jnp.concatenate(
        [x_ref[...].T.astype(F8),
         (dvb.T * DS).astype(F8),
         jnp.ones((1, BN1), F8)], axis=0)
    acc_ref[...] += jax.lax.dot_general(
        lhs, incq, _NT, preferred_element_type=jnp.float32)


def _layer_sweep(incq_ref, x_ref, dv_ref, acc_ref, wl_ref, wo_ref, b_ref,
                 out_ref, m2t_ref, y_ref):
    p = pl.program_id(0)
    i = pl.program_id(1)
    crow = jax.lax.rsqrt(acc_ref[FEATS:FEATS + 1, :] * (1.0 / DS)
                         * acc_ref[FEATS + 1:FEATS + 2, :])

    @pl.when((p == 0) & (i == 0))
    def _start_l1():
        y_ref[...] = (acc_ref[0:FEATS, :] * (crow * YS)).T.astype(F8)
        m2t_ref[...] = jnp.zeros_like(m2t_ref)

    @pl.when((p == 1) & (i == 0))
    def _start_l2():
        y_ref[...] = (m2t_ref[...] * (crow * YS)).T.astype(F8)

    x0p = jax.lax.dot_general(
        incq_ref[...], y_ref[...], _NT, preferred_element_type=jnp.float32)
    xcombt = (((1.0 - ALPHA) / YS) * x0p.T * jax.lax.rsqrt(dv_ref[...].T)
              + ALPHA * x_ref[...].T)
    xlt = (1.0 - BETA) * xcombt + BETA * jax.lax.dot_general(
        wl_ref[p], xcombt, _NT, preferred_element_type=jnp.float32)

    @pl.when(p == 0)
    def _acc_m2():
        m2t_ref[...] += jax.lax.dot_general(
            xlt.astype(F8), incq_ref[...], _NT,
            preferred_element_type=jnp.float32)

    @pl.when(p == 1)
    def _head():
        outt = jax.lax.dot_general(
            wo_ref[...], xlt, _NT, preferred_element_type=jnp.float32)
        out_ref[...] = outt.T + b_ref[...]


def kernel(x_0, incidence_1, W_layers, W_out, b_out):
    n, e, f = N_NODES, N_EDGES, FEATS
    full = lambda shape: pl.BlockSpec(shape, lambda *_: (0,) * len(shape))

    acc, dv, inc_q = pl.pallas_call(
        _stats_sweep,
        grid=(n // BN1,),
        in_specs=[pl.BlockSpec((BN1, f), lambda i: (i, 0)),
                  pl.BlockSpec((BN1, e), lambda i: (i, 0))],
        out_specs=[full((f + 2, e)),
                   pl.BlockSpec((BN1, 1), lambda i: (i, 0)),
                   pl.BlockSpec((BN1, e), lambda i: (i, 0))],
        out_shape=[
            jax.ShapeDtypeStruct((f + 2, e), jnp.float32),
            jax.ShapeDtypeStruct((n, 1), jnp.float32),
            jax.ShapeDtypeStruct((n, e), F8),
        ],
    )(x_0, incidence_1)

    out = pl.pallas_call(
        _layer_sweep,
        grid=(2, n // BN2),
        in_specs=[pl.BlockSpec((BN2, e), lambda p, i: (i, 0)),
                  pl.BlockSpec((BN2, f), lambda p, i: (i, 0)),
                  pl.BlockSpec((BN2, 1), lambda p, i: (i, 0)),
                  full((f + 2, e)), full((2, f, f)), full((f, f)),
                  full((1, f))],
        out_specs=pl.BlockSpec((BN2, f), lambda p, i: (i, 0)),
        out_shape=jax.ShapeDtypeStruct((n, f), jnp.float32),
        scratch_shapes=[pltpu.VMEM((f, e), jnp.float32),
                        pltpu.VMEM((e, f), F8)],
    )(inc_q, x_0, dv, acc, W_layers, W_out, b_out[None, :])

    return out
